# baseline (speedup 1.0000x reference)
import jax
import jax.numpy as jnp
from jax.experimental import pallas as pl
from jax.experimental.pallas import tpu as pltpu


def _body(l_ref, f_ref, fm_ref, w1t_ref, w2t_ref, b_ref, out_ref):
    x = jnp.dot(
        l_ref[...].astype(jnp.bfloat16),
        f_ref[...],
        preferred_element_type=jnp.float32,
    )
    f = fm_ref[...]
    out_ref[...] = (
        jnp.dot(f + x, w1t_ref[...], preferred_element_type=jnp.float32)
        + jnp.dot(x * f, w2t_ref[...], preferred_element_type=jnp.float32)
        + b_ref[...]
    )


def kernel(lap_matrix, eye_matrix, features, W1, b1, W2, b2):
    n, d = features.shape
    bm = min(256, n)
    nm = pl.cdiv(n, bm)

    bias = (b1 + b2).reshape(1, d)
    f_bf16 = features.astype(jnp.bfloat16)

    in_specs = [
        pl.BlockSpec((bm, n), lambda i: (i, 0)),
        pl.BlockSpec((n, d), lambda i: (0, 0)),
        pl.BlockSpec((bm, d), lambda i: (i, 0)),
        pl.BlockSpec((d, d), lambda i: (0, 0)),
        pl.BlockSpec((d, d), lambda i: (0, 0)),
        pl.BlockSpec((1, d), lambda i: (0, 0)),
    ]

    return pl.pallas_call(
        _body,
        grid=(nm,),
        in_specs=in_specs,
        out_specs=pl.BlockSpec((bm, d), lambda i: (i, 0)),
        out_shape=jax.ShapeDtypeStruct((n, d), jnp.float32),
        compiler_params=pltpu.CompilerParams(
            dimension_semantics=("arbitrary",),
            vmem_limit_bytes=63 * 1024 * 1024,
        ),
    )(lap_matrix, f_bf16, features, W1.T, W2.T, bias)


# speedup vs baseline: 1.0144x; 1.0144x over previous
import functools

import jax
import jax.numpy as jnp
from jax.experimental import pallas as pl
from jax.experimental.pallas import tpu as pltpu


def _body(nm, nbuf, bm, l_hbm, f_ref, fm_ref, w1t_ref, w2t_ref, b_ref, out_ref,
          lbuf, sems):
    i = pl.program_id(0)

    @pl.when(i == 0)
    def _():
        for j in range(nbuf):
            pltpu.make_async_copy(
                l_hbm.at[pl.ds(j * bm, bm), :], lbuf.at[j], sems.at[j]
            ).start()

    slot = jax.lax.rem(i, nbuf)
    pltpu.make_async_copy(
        l_hbm.at[pl.ds(i * bm, bm), :], lbuf.at[slot], sems.at[slot]
    ).wait()

    x = jnp.dot(
        lbuf[slot].astype(jnp.bfloat16),
        f_ref[...].astype(jnp.bfloat16),
        preferred_element_type=jnp.float32,
    )
    f = fm_ref[...]
    out_ref[...] = (
        jnp.dot(f + x, w1t_ref[...], preferred_element_type=jnp.float32)
        + jnp.dot(x * f, w2t_ref[...], preferred_element_type=jnp.float32)
        + b_ref[...]
    )

    nxt = i + nbuf

    @pl.when(nxt < nm)
    def _():
        pltpu.make_async_copy(
            l_hbm.at[pl.ds(nxt * bm, bm), :], lbuf.at[slot], sems.at[slot]
        ).start()


def kernel(lap_matrix, eye_matrix, features, W1, b1, W2, b2):
    n, d = features.shape
    bm = min(256, n)
    nm = n // bm
    nbuf = min(3, nm)

    bias = (b1 + b2).reshape(1, d)

    in_specs = [
        pl.BlockSpec(memory_space=pltpu.HBM),
        pl.BlockSpec((n, d), lambda i: (0, 0)),
        pl.BlockSpec((bm, d), lambda i: (i, 0)),
        pl.BlockSpec((d, d), lambda i: (0, 0)),
        pl.BlockSpec((d, d), lambda i: (0, 0)),
        pl.BlockSpec((1, d), lambda i: (0, 0)),
    ]

    return pl.pallas_call(
        functools.partial(_body, nm, nbuf, bm),
        grid=(nm,),
        in_specs=in_specs,
        out_specs=pl.BlockSpec((bm, d), lambda i: (i, 0)),
        out_shape=jax.ShapeDtypeStruct((n, d), jnp.float32),
        scratch_shapes=[
            pltpu.VMEM((nbuf, bm, n), jnp.float32),
            pltpu.SemaphoreType.DMA((nbuf,)),
        ],
        compiler_params=pltpu.CompilerParams(
            dimension_semantics=("arbitrary",),
            vmem_limit_bytes=63 * 1024 * 1024,
        ),
    )(lap_matrix, features, features, W1.T, W2.T, bias)


# speedup vs baseline: 1.0148x; 1.0003x over previous
import jax
import jax.numpy as jnp
from jax.experimental import pallas as pl
from jax.experimental.pallas import tpu as pltpu


def _body(l_ref, f_ref, fm_ref, w1t_ref, w2t_ref, b_ref, out_ref):
    x = jnp.dot(
        l_ref[...].astype(jnp.bfloat16),
        f_ref[...].astype(jnp.bfloat16),
        preferred_element_type=jnp.float32,
    )
    f = fm_ref[...]
    out_ref[...] = (
        jnp.dot(f + x, w1t_ref[...], preferred_element_type=jnp.float32)
        + jnp.dot(x * f, w2t_ref[...], preferred_element_type=jnp.float32)
        + b_ref[...]
    )


def kernel(lap_matrix, eye_matrix, features, W1, b1, W2, b2):
    n, d = features.shape
    bm = min(384, n)
    nm = pl.cdiv(n, bm)

    bias = (b1 + b2).reshape(1, d)

    in_specs = [
        pl.BlockSpec((bm, n), lambda i: (i, 0)),
        pl.BlockSpec((n, d), lambda i: (0, 0)),
        pl.BlockSpec((bm, d), lambda i: (i, 0)),
        pl.BlockSpec((d, d), lambda i: (0, 0)),
        pl.BlockSpec((d, d), lambda i: (0, 0)),
        pl.BlockSpec((1, d), lambda i: (0, 0)),
    ]

    return pl.pallas_call(
        _body,
        grid=(nm,),
        in_specs=in_specs,
        out_specs=pl.BlockSpec((bm, d), lambda i: (i, 0)),
        out_shape=jax.ShapeDtypeStruct((n, d), jnp.float32),
        compiler_params=pltpu.CompilerParams(
            dimension_semantics=("arbitrary",),
            vmem_limit_bytes=63 * 1024 * 1024,
        ),
    )(lap_matrix, features, features, W1.T, W2.T, bias)


# speedup vs baseline: 1.0443x; 1.0291x over previous
import functools

import jax
import jax.numpy as jnp
from jax.experimental import pallas as pl
from jax.experimental.pallas import tpu as pltpu

_SPLIT = 2
_T = (((1,), (1,)), ((), ()))


def _body(sub, l0_ref, l1_ref, f_ref, w1_ref, w2_ref, b1_ref, b2_ref, out_ref):
    i = pl.program_id(0)
    fb = f_ref[...].astype(jnp.bfloat16)
    w1 = w1_ref[...]
    w2 = w2_ref[...]
    b = b1_ref[...] + b2_ref[...]
    for j, l_ref in enumerate((l0_ref, l1_ref)):
        x = jnp.dot(
            l_ref[...].astype(jnp.bfloat16),
            fb,
            preferred_element_type=jnp.float32,
        )
        f = f_ref[pl.ds(i * _SPLIT * sub + j * sub, sub), :]
        out_ref[pl.ds(j * sub, sub), :] = (
            jax.lax.dot_general(f + x, w1, _T, preferred_element_type=jnp.float32)
            + jax.lax.dot_general(x * f, w2, _T, preferred_element_type=jnp.float32)
            + b
        )


def kernel(lap_matrix, eye_matrix, features, W1, b1, W2, b2):
    n, d = features.shape
    bm = min(256, n)
    sub = bm // _SPLIT
    nm = n // bm

    l_specs = [
        pl.BlockSpec((sub, n), functools.partial(lambda j, i: (_SPLIT * i + j, 0), j))
        for j in range(_SPLIT)
    ]
    in_specs = l_specs + [
        pl.BlockSpec((n, d), lambda i: (0, 0)),
        pl.BlockSpec((d, d), lambda i: (0, 0)),
        pl.BlockSpec((d, d), lambda i: (0, 0)),
        pl.BlockSpec((1, d), lambda i: (0, 0)),
        pl.BlockSpec((1, d), lambda i: (0, 0)),
    ]

    return pl.pallas_call(
        functools.partial(_body, sub),
        grid=(nm,),
        in_specs=in_specs,
        out_specs=pl.BlockSpec((bm, d), lambda i: (i, 0)),
        out_shape=jax.ShapeDtypeStruct((n, d), jnp.float32),
        compiler_params=pltpu.CompilerParams(
            dimension_semantics=("arbitrary",),
            vmem_limit_bytes=63 * 1024 * 1024,
        ),
    )(lap_matrix, lap_matrix, features, W1, W2,
      b1.reshape(1, d), b2.reshape(1, d))
